# baseline (speedup 1.0000x reference)
import functools

import jax
import jax.numpy as jnp
from jax import lax
from jax.experimental import pallas as pl
from jax.experimental.pallas import tpu as pltpu
from jax.experimental.pallas import tpu_sc as plsc

BATCH = 16384
EMBD_DIM = 128
GENRE_SIZE = 128
GATHER_WINDOW = 128
TC_BLOCK = 4096

_vector_mesh = plsc.VectorSubcoreMesh(
    core_axis_name="core", subcore_axis_name="subcore"
)


N_WORKERS = 32


def _sc_gather(embd_table, idx2d, nrows, win0=0):
    rows_per_w = nrows // N_WORKERS
    n_win = rows_per_w // GATHER_WINDOW

    @functools.partial(
        pl.kernel,
        out_type=jax.ShapeDtypeStruct((nrows, EMBD_DIM), jnp.float32),
        mesh=_vector_mesh,
        scratch_types=[
            pltpu.VMEM((n_win, GATHER_WINDOW), jnp.int32),
            pltpu.VMEM((rows_per_w, EMBD_DIM), jnp.float32),
            pltpu.SemaphoreType.DMA((n_win,)),
            pltpu.SemaphoreType.DMA,
        ],
    )
    def gather_kernel(table_hbm, idx_hbm, out_hbm, idx_v, rows_v, gsems, osem):
        wid = lax.axis_index("subcore") * 2 + lax.axis_index("core")
        pltpu.sync_copy(idx_hbm.at[pl.ds(win0 + wid * n_win, n_win)], idx_v)
        gathers = [
            pltpu.async_copy(
                table_hbm.at[idx_v.at[j]],
                rows_v.at[pl.ds(j * GATHER_WINDOW, GATHER_WINDOW)],
                gsems.at[j],
            )
            for j in range(n_win)
        ]
        writes = []
        for j in range(n_win):
            gathers[j].wait()
            writes.append(
                pltpu.async_copy(
                    rows_v.at[pl.ds(j * GATHER_WINDOW, GATHER_WINDOW)],
                    out_hbm.at[
                        pl.ds(wid * rows_per_w + j * GATHER_WINDOW, GATHER_WINDOW)
                    ],
                    osem,
                )
            )
        for cp in writes:
            cp.wait()

    return gather_kernel(embd_table, idx2d)


def _linear_body(x_ref, w_ref, b_ref, o_ref):
    y = lax.dot_general(
        x_ref[...].astype(jnp.bfloat16),
        w_ref[...].astype(jnp.bfloat16),
        (((1,), (1,)), ((), ())),
        preferred_element_type=jnp.float32,
    )
    o_ref[...] = jnp.maximum(y + b_ref[...], 0.0)


def _linear_body_aliased(x_ref, w_ref, b_ref, prev_ref, o_ref):
    del prev_ref
    _linear_body(x_ref, w_ref, b_ref, o_ref)


def _tc_linear_chunk(x, W, b2, block0, prev=None):
    nb = x.shape[0] // TC_BLOCK
    out_shape = jax.ShapeDtypeStruct((BATCH, GENRE_SIZE), jnp.float32)
    in_specs = [
        pl.BlockSpec((TC_BLOCK, EMBD_DIM), lambda i: (i, 0)),
        pl.BlockSpec((EMBD_DIM, GENRE_SIZE), lambda i: (0, 0)),
        pl.BlockSpec((1, GENRE_SIZE), lambda i: (0, 0)),
    ]
    out_specs = pl.BlockSpec(
        (TC_BLOCK, GENRE_SIZE), lambda i: (block0 + i, 0)
    )
    if prev is None:
        return pl.pallas_call(
            _linear_body,
            grid=(nb,),
            in_specs=in_specs,
            out_specs=out_specs,
            out_shape=out_shape,
        )(x, W, b2)
    in_specs.append(pl.BlockSpec((8, GENRE_SIZE), lambda i: (0, 0)))
    return pl.pallas_call(
        _linear_body_aliased,
        grid=(nb,),
        in_specs=in_specs,
        out_specs=out_specs,
        out_shape=out_shape,
        input_output_aliases={3: 0},
    )(x, W, b2, prev)


N_CHUNKS = 2
CHUNK = BATCH // N_CHUNKS


def kernel(item, embd_table, W, b):
    b2 = b.reshape(1, GENRE_SIZE)
    idx2d = item.reshape(BATCH // GATHER_WINDOW, GATHER_WINDOW)
    chunk_wins = CHUNK // GATHER_WINDOW
    xs = [
        _sc_gather(embd_table, idx2d, CHUNK, win0=k * chunk_wins)
        for k in range(N_CHUNKS)
    ]
    out = None
    for k in range(N_CHUNKS):
        out = _tc_linear_chunk(xs[k], W, b2, k * (CHUNK // TC_BLOCK), out)
    return out


# speedup vs baseline: 1.1208x; 1.1208x over previous
import functools

import jax
import jax.numpy as jnp
from jax import lax
from jax.experimental import pallas as pl
from jax.experimental.pallas import tpu as pltpu
from jax.experimental.pallas import tpu_sc as plsc

BATCH = 16384
EMBD_DIM = 128
GENRE_SIZE = 128
GATHER_WINDOW = 128
TC_BLOCK = 8192

_vector_mesh = plsc.VectorSubcoreMesh(
    core_axis_name="core", subcore_axis_name="subcore"
)


N_WORKERS = 32


def _sc_gather(embd_table, idx2d, nrows, win0=0):
    rows_per_w = nrows // N_WORKERS
    n_win = rows_per_w // GATHER_WINDOW

    @functools.partial(
        pl.kernel,
        out_type=jax.ShapeDtypeStruct((nrows, EMBD_DIM), jnp.float32),
        mesh=_vector_mesh,
        scratch_types=[
            pltpu.VMEM((n_win, GATHER_WINDOW), jnp.int32),
            pltpu.VMEM((rows_per_w, EMBD_DIM), jnp.float32),
            pltpu.SemaphoreType.DMA((n_win,)),
            pltpu.SemaphoreType.DMA,
        ],
    )
    def gather_kernel(table_hbm, idx_hbm, out_hbm, idx_v, rows_v, gsems, osem):
        wid = lax.axis_index("subcore") * 2 + lax.axis_index("core")
        pltpu.sync_copy(idx_hbm.at[pl.ds(win0 + wid * n_win, n_win)], idx_v)
        gathers = [
            pltpu.async_copy(
                table_hbm.at[idx_v.at[j]],
                rows_v.at[pl.ds(j * GATHER_WINDOW, GATHER_WINDOW)],
                gsems.at[j],
            )
            for j in range(n_win)
        ]
        writes = []
        for j in range(n_win):
            gathers[j].wait()
            writes.append(
                pltpu.async_copy(
                    rows_v.at[pl.ds(j * GATHER_WINDOW, GATHER_WINDOW)],
                    out_hbm.at[
                        pl.ds(wid * rows_per_w + j * GATHER_WINDOW, GATHER_WINDOW)
                    ],
                    osem,
                )
            )
        for cp in writes:
            cp.wait()

    return gather_kernel(embd_table, idx2d)


def _linear_body(x_ref, w_ref, b_ref, o_ref):
    y = lax.dot_general(
        x_ref[...].astype(jnp.bfloat16),
        w_ref[...].astype(jnp.bfloat16),
        (((1,), (1,)), ((), ())),
        preferred_element_type=jnp.float32,
    )
    o_ref[...] = jnp.maximum(y + b_ref[...], 0.0)


def _linear_body_aliased(x_ref, w_ref, b_ref, prev_ref, o_ref):
    del prev_ref
    _linear_body(x_ref, w_ref, b_ref, o_ref)


def _tc_linear_chunk(x, W, b2, block0, prev=None):
    nb = x.shape[0] // TC_BLOCK
    out_shape = jax.ShapeDtypeStruct((BATCH, GENRE_SIZE), jnp.float32)
    in_specs = [
        pl.BlockSpec((TC_BLOCK, EMBD_DIM), lambda i: (i, 0)),
        pl.BlockSpec((EMBD_DIM, GENRE_SIZE), lambda i: (0, 0)),
        pl.BlockSpec((1, GENRE_SIZE), lambda i: (0, 0)),
    ]
    out_specs = pl.BlockSpec(
        (TC_BLOCK, GENRE_SIZE), lambda i: (block0 + i, 0)
    )
    if prev is None:
        return pl.pallas_call(
            _linear_body,
            grid=(nb,),
            in_specs=in_specs,
            out_specs=out_specs,
            out_shape=out_shape,
        )(x, W, b2)
    in_specs.append(pl.BlockSpec((8, GENRE_SIZE), lambda i: (0, 0)))
    return pl.pallas_call(
        _linear_body_aliased,
        grid=(nb,),
        in_specs=in_specs,
        out_specs=out_specs,
        out_shape=out_shape,
        input_output_aliases={3: 0},
    )(x, W, b2, prev)


N_CHUNKS = 1
CHUNK = BATCH // N_CHUNKS


def kernel(item, embd_table, W, b):
    b2 = b.reshape(1, GENRE_SIZE)
    idx2d = item.reshape(BATCH // GATHER_WINDOW, GATHER_WINDOW)
    chunk_wins = CHUNK // GATHER_WINDOW
    xs = [
        _sc_gather(embd_table, idx2d, CHUNK, win0=k * chunk_wins)
        for k in range(N_CHUNKS)
    ]
    out = None
    for k in range(N_CHUNKS):
        out = _tc_linear_chunk(xs[k], W, b2, k * (CHUNK // TC_BLOCK), out)
    return out
